# baseline (device time: 19393 ns/iter reference)
import jax
import jax.numpy as jnp
from jax import lax
from jax.experimental import pallas as pl
from jax.experimental.pallas import tpu as pltpu

N_DEV = 4
NSUB = 2


def kernel(A, B):
    m, _ = A.shape
    _, n = B.shape
    QR = m // 4
    HR = QR // NSUB

    def body(a_ref, b_ref, out_ref, w_ref, comm_ref, send_sems, recv_sems):
        my_pos = lax.axis_index("i")
        p1 = my_pos ^ 1
        p2 = 3 - my_pos

        k1 = jnp.where((my_pos == 0) | (my_pos == 3), 0, 1)
        o1 = 1 - k1
        k2 = jnp.where(my_pos <= 1, 2, 3)
        o2 = 5 - k2

        barrier_sem = pltpu.get_barrier_semaphore()
        for nbr in [p1, p2]:
            pl.semaphore_signal(
                barrier_sem, inc=1,
                device_id=(nbr,), device_id_type=pl.DeviceIdType.MESH,
            )
        pl.semaphore_wait(barrier_sem, 2)

        def sub(q, s):
            return pl.ds(q * QR + s * HR, HR)

        def mm_sub(q, s):
            w_ref[sub(q, s), :] = jnp.dot(
                a_ref[sub(q, s), :], b_ref[:, :],
                preferred_element_type=jnp.float32,
            ).astype(jnp.bfloat16)

        def rdma_to_slot(q, s, partner, slot, t):
            return pltpu.make_async_remote_copy(
                src_ref=w_ref.at[sub(q, s), :],
                dst_ref=comm_ref.at[slot],
                send_sem=send_sems.at[t],
                recv_sem=recv_sems.at[t],
                device_id=(partner,),
                device_id_type=pl.DeviceIdType.MESH,
            )

        def rdma_to_w(q, s, partner, t):
            return pltpu.make_async_remote_copy(
                src_ref=w_ref.at[sub(q, s), :],
                dst_ref=w_ref.at[sub(q, s), :],
                send_sem=send_sems.at[t],
                recv_sem=recv_sems.at[t],
                device_id=(partner,),
                device_id_type=pl.DeviceIdType.MESH,
            )

        mm_sub(o1, 0)
        s1 = [None] * 4
        s1[0] = rdma_to_slot(o1, 0, p1, 0, 0)
        s1[0].start()
        mm_sub(o2, 0)
        s1[1] = rdma_to_slot(o2, 0, p2, 1, 1)
        s1[1].start()
        mm_sub(o1, 1)
        s1[2] = rdma_to_slot(o1, 1, p1, 2, 2)
        s1[2].start()
        mm_sub(o2, 1)
        s1[3] = rdma_to_slot(o2, 1, p2, 3, 3)
        s1[3].start()
        mm_sub(k1, 0)
        mm_sub(k2, 0)
        mm_sub(k1, 1)
        mm_sub(k2, 1)

        s2 = [None] * 4
        s1[0].wait_recv()
        w_ref[sub(k1, 0), :] += comm_ref[0]
        s2[0] = rdma_to_slot(k1, 0, p2, 4, 4)
        s2[0].start()
        s1[1].wait_recv()
        w_ref[sub(k2, 0), :] += comm_ref[1]
        s2[1] = rdma_to_slot(k2, 0, p1, 5, 5)
        s2[1].start()
        s1[2].wait_recv()
        w_ref[sub(k1, 1), :] += comm_ref[2]
        s2[2] = rdma_to_slot(k1, 1, p2, 6, 6)
        s2[2].start()
        s1[3].wait_recv()
        w_ref[sub(k2, 1), :] += comm_ref[3]
        s2[3] = rdma_to_slot(k2, 1, p1, 7, 7)
        s2[3].start()

        s3 = [None] * 4
        s2[0].wait_recv()
        w_ref[sub(k1, 0), :] += comm_ref[4]
        s3[0] = rdma_to_w(k1, 0, p1, 8)
        s3[0].start()
        s2[1].wait_recv()
        w_ref[sub(k2, 0), :] += comm_ref[5]
        s3[1] = rdma_to_w(k2, 0, p2, 9)
        s3[1].start()
        s2[2].wait_recv()
        w_ref[sub(k1, 1), :] += comm_ref[6]
        s3[2] = rdma_to_w(k1, 1, p1, 10)
        s3[2].start()
        s2[3].wait_recv()
        w_ref[sub(k2, 1), :] += comm_ref[7]
        s3[3] = rdma_to_w(k2, 1, p2, 11)
        s3[3].start()

        out_ref[pl.ds(k1 * QR, QR), :] = w_ref[pl.ds(k1 * QR, QR), :].astype(
            jnp.float32
        )
        out_ref[pl.ds(k2 * QR, QR), :] = w_ref[pl.ds(k2 * QR, QR), :].astype(
            jnp.float32
        )

        s3[0].wait_recv()
        s3[2].wait_recv()
        out_ref[pl.ds(o1 * QR, QR), :] = w_ref[pl.ds(o1 * QR, QR), :].astype(
            jnp.float32
        )
        s3[1].wait_recv()
        s3[3].wait_recv()
        out_ref[pl.ds(o2 * QR, QR), :] = w_ref[pl.ds(o2 * QR, QR), :].astype(
            jnp.float32
        )

        for r in s1 + s2 + s3:
            r.wait_send()

    return pl.pallas_call(
        body,
        out_shape=jax.ShapeDtypeStruct((m, n), jnp.float32),
        in_specs=[
            pl.BlockSpec(memory_space=pltpu.VMEM),
            pl.BlockSpec(memory_space=pltpu.VMEM),
        ],
        out_specs=pl.BlockSpec(memory_space=pltpu.VMEM),
        scratch_shapes=[
            pltpu.VMEM((m, n), jnp.bfloat16),
            pltpu.VMEM((8, HR, n), jnp.bfloat16),
            pltpu.SemaphoreType.DMA((12,)),
            pltpu.SemaphoreType.DMA((12,)),
        ],
        compiler_params=pltpu.CompilerParams(collective_id=0),
    )(A, B)


# device time: 19289 ns/iter; 1.0054x vs baseline; 1.0054x over previous
import jax
import jax.numpy as jnp
from jax import lax
from jax.experimental import pallas as pl
from jax.experimental.pallas import tpu as pltpu

N_DEV = 4
NSUB = 4


def kernel(A, B):
    m, _ = A.shape
    _, n = B.shape
    QR = m // 4
    HR = QR // NSUB

    def body(a_ref, b_ref, out_ref, w_ref, comm_ref, send_sems, recv_sems):
        my_pos = lax.axis_index("i")
        p1 = my_pos ^ 1
        p2 = 3 - my_pos

        k1 = jnp.where((my_pos == 0) | (my_pos == 3), 0, 1)
        o1 = 1 - k1
        k2 = jnp.where(my_pos <= 1, 2, 3)
        o2 = 5 - k2

        sched = [(o1, k1, p1, p2), (o2, k2, p2, p1)]

        barrier_sem = pltpu.get_barrier_semaphore()
        for nbr in [p1, p2]:
            pl.semaphore_signal(
                barrier_sem, inc=1,
                device_id=(nbr,), device_id_type=pl.DeviceIdType.MESH,
            )
        pl.semaphore_wait(barrier_sem, 2)

        def sub(q, j):
            return pl.ds(q * QR + j * HR, HR)

        def mm_sub(q, j):
            w_ref[sub(q, j), :] = jnp.dot(
                a_ref[sub(q, j), :], b_ref[:, :],
                preferred_element_type=jnp.float32,
            ).astype(jnp.bfloat16)

        def rdma_to_slot(q, j, partner, slot, t):
            return pltpu.make_async_remote_copy(
                src_ref=w_ref.at[sub(q, j), :],
                dst_ref=comm_ref.at[slot],
                send_sem=send_sems.at[t],
                recv_sem=recv_sems.at[t],
                device_id=(partner,),
                device_id_type=pl.DeviceIdType.MESH,
            )

        def rdma_w_to_w(q_src, q_dst, j, partner, t):
            return pltpu.make_async_remote_copy(
                src_ref=w_ref.at[sub(q_src, j), :],
                dst_ref=w_ref.at[sub(q_dst, j), :],
                send_sem=send_sems.at[t],
                recv_sem=recv_sems.at[t],
                device_id=(partner,),
                device_id_type=pl.DeviceIdType.MESH,
            )

        s1 = [[None] * 2 for _ in range(NSUB)]
        s2 = [[None] * 2 for _ in range(NSUB)]
        s3_send = [[None] * 2 for _ in range(NSUB)]
        s3_recv = [[None] * 2 for _ in range(NSUB)]

        for j in range(NSUB):
            for c, (o, _, pa, _) in enumerate(sched):
                mm_sub(o, j)
                t = 2 * j + c
                s1[j][c] = rdma_to_slot(o, j, pa, t, t)
                s1[j][c].start()
        for j in range(NSUB):
            for _, k, _, _ in sched:
                mm_sub(k, j)

        for j in range(NSUB):
            for c, (_, k, _, pb) in enumerate(sched):
                s1[j][c].wait_recv()
                w_ref[sub(k, j), :] += comm_ref[2 * j + c]
                t = 2 * NSUB + 2 * j + c
                s2[j][c] = rdma_to_slot(k, j, pb, t, t)
                s2[j][c].start()

        for j in range(NSUB):
            for c, (o, k, pa, _) in enumerate(sched):
                s2[j][c].wait_recv()
                w_ref[sub(k, j), :] += comm_ref[2 * NSUB + 2 * j + c]
                t = 4 * NSUB + 2 * j + c
                s3_send[j][c] = rdma_w_to_w(k, k, j, pa, t)
                s3_send[j][c].start()
                s3_recv[j][c] = rdma_w_to_w(o, o, j, pa, t)

        out_ref[pl.ds(k1 * QR, QR), :] = w_ref[pl.ds(k1 * QR, QR), :].astype(
            jnp.float32
        )
        out_ref[pl.ds(k2 * QR, QR), :] = w_ref[pl.ds(k2 * QR, QR), :].astype(
            jnp.float32
        )

        for c, (o, _, _, _) in enumerate(sched):
            for j in range(NSUB):
                s3_recv[j][c].wait_recv()
            out_ref[pl.ds(o * QR, QR), :] = w_ref[pl.ds(o * QR, QR), :].astype(
                jnp.float32
            )

        for j in range(NSUB):
            for c in range(2):
                s1[j][c].wait_send()
                s2[j][c].wait_send()
                s3_send[j][c].wait_send()

    return pl.pallas_call(
        body,
        out_shape=jax.ShapeDtypeStruct((m, n), jnp.float32),
        in_specs=[
            pl.BlockSpec(memory_space=pltpu.VMEM),
            pl.BlockSpec(memory_space=pltpu.VMEM),
        ],
        out_specs=pl.BlockSpec(memory_space=pltpu.VMEM),
        scratch_shapes=[
            pltpu.VMEM((m, n), jnp.bfloat16),
            pltpu.VMEM((4 * NSUB, HR, n), jnp.bfloat16),
            pltpu.SemaphoreType.DMA((6 * NSUB,)),
            pltpu.SemaphoreType.DMA((6 * NSUB,)),
        ],
        compiler_params=pltpu.CompilerParams(collective_id=0),
    )(A, B)
